# baseline (speedup 1.0000x reference)
import functools

import jax
import jax.numpy as jnp
from jax.experimental import pallas as pl
from jax.experimental.pallas import tpu as pltpu

N = 2048
NUM_EDGE = 5
NUM_CHANNELS = 2
IN_DIM = 256
HIDDEN = 64
NUM_CLASS = 8
WIDE = 384
BM = 256


def _combine(filt_ref, a_ref, s, c):
    acc = filt_ref[s, c, 0] * a_ref[0]
    for r in range(1, NUM_EDGE):
        acc = acc + filt_ref[s, c, r] * a_ref[r]
    return acc


def _first_kernel(filt_ref, a_ref, t_ref, out_ref, q1_ref, q0_ref):
    for c in range(NUM_CHANNELS):
        ac = _combine(filt_ref, a_ref, 2, c)
        out_ref[c] = jnp.dot(ac, t_ref[c], preferred_element_type=jnp.float32)
        q1_ref[c] = _combine(filt_ref, a_ref, 1, c).astype(jnp.bfloat16)
        q0_ref[c] = _combine(filt_ref, a_ref, 0, c).astype(jnp.bfloat16)


def _stage_kernel(q_ref, t_ref, out_ref):
    for c in range(NUM_CHANNELS):
        out_ref[c] = jnp.dot(q_ref[c], t_ref[c].astype(jnp.bfloat16),
                             preferred_element_type=jnp.float32)


def _final_kernel(q_ref, t_ref, gw_ref, gb_ref, l1w_ref, l1b_ref,
                  l2w_ref, l2b_ref, y_ref):
    xs = []
    for c in range(NUM_CHANNELS):
        v = jnp.dot(q_ref[c], t_ref[c].astype(jnp.bfloat16),
                    preferred_element_type=jnp.float32)
        num = v[:, :IN_DIM]
        e = v[:, IN_DIM:IN_DIM + 1]
        agg = num / jnp.where(e == 0.0, 1.0, e)
        x = jnp.dot(agg, gw_ref[...], preferred_element_type=jnp.float32)
        xs.append(jnp.maximum(x + gb_ref[...], 0.0))
    z = (jnp.dot(xs[0], l1w_ref[:HIDDEN], preferred_element_type=jnp.float32)
         + jnp.dot(xs[1], l1w_ref[HIDDEN:], preferred_element_type=jnp.float32)
         + l1b_ref[...])
    z = jnp.maximum(z, 0.0)
    y_ref[...] = (jnp.dot(z, l2w_ref[...], preferred_element_type=jnp.float32)
                  + l2b_ref[...])


def _first(filt, A, t):
    return pl.pallas_call(
        _first_kernel,
        grid=(N // BM,),
        in_specs=[
            pl.BlockSpec(memory_space=pltpu.SMEM),
            pl.BlockSpec((NUM_EDGE, BM, N), lambda i: (0, i, 0)),
            pl.BlockSpec((NUM_CHANNELS, N, WIDE), lambda i: (0, 0, 0)),
        ],
        out_specs=[
            pl.BlockSpec((NUM_CHANNELS, BM, WIDE), lambda i: (0, i, 0)),
            pl.BlockSpec((NUM_CHANNELS, BM, N), lambda i: (0, i, 0)),
            pl.BlockSpec((NUM_CHANNELS, BM, N), lambda i: (0, i, 0)),
        ],
        out_shape=[
            jax.ShapeDtypeStruct((NUM_CHANNELS, N, WIDE), jnp.float32),
            jax.ShapeDtypeStruct((NUM_CHANNELS, N, N), jnp.bfloat16),
            jax.ShapeDtypeStruct((NUM_CHANNELS, N, N), jnp.bfloat16),
        ],
    )(filt, A, t)


def _stage(Q, t):
    return pl.pallas_call(
        _stage_kernel,
        grid=(N // BM,),
        in_specs=[
            pl.BlockSpec((NUM_CHANNELS, BM, N), lambda i: (0, i, 0)),
            pl.BlockSpec((NUM_CHANNELS, N, WIDE), lambda i: (0, 0, 0)),
        ],
        out_specs=pl.BlockSpec((NUM_CHANNELS, BM, WIDE), lambda i: (0, i, 0)),
        out_shape=jax.ShapeDtypeStruct((NUM_CHANNELS, N, WIDE), jnp.float32),
    )(Q, t)


def _final(Q, t, gcn_w, gcn_b, lin1_w, lin1_b, lin2_w, lin2_b):
    small = lambda shp: pl.BlockSpec(shp, lambda i: tuple(0 for _ in shp))
    return pl.pallas_call(
        _final_kernel,
        grid=(N // BM,),
        in_specs=[
            pl.BlockSpec((NUM_CHANNELS, BM, N), lambda i: (0, i, 0)),
            pl.BlockSpec((NUM_CHANNELS, N, WIDE), lambda i: (0, 0, 0)),
            small((IN_DIM, HIDDEN)),
            small((1, HIDDEN)),
            small((NUM_CHANNELS * HIDDEN, HIDDEN)),
            small((1, HIDDEN)),
            small((HIDDEN, NUM_CLASS)),
            small((1, NUM_CLASS)),
        ],
        out_specs=pl.BlockSpec((BM, NUM_CLASS), lambda i: (i, 0)),
        out_shape=jax.ShapeDtypeStruct((N, NUM_CLASS), jnp.float32),
    )(Q, t, gcn_w, gcn_b, lin1_w, lin1_b, lin2_w, lin2_b)


def kernel(A, h, W_conv, gcn_w, gcn_b, lin1_w, lin1_b, lin2_w, lin2_b):
    filt = jax.nn.softmax(W_conv, axis=2)
    t0 = jnp.concatenate(
        [h, jnp.ones((N, 1), jnp.float32), jnp.zeros((N, WIDE - IN_DIM - 1), jnp.float32)],
        axis=1)
    t = jnp.stack([t0] * NUM_CHANNELS)
    t, Q1, Q0 = _first(filt, A, t)
    t = _stage(Q1, t)
    return _final(Q0, t,
                  gcn_w, gcn_b.reshape(1, HIDDEN),
                  lin1_w, lin1_b.reshape(1, HIDDEN),
                  lin2_w, lin2_b.reshape(1, NUM_CLASS))


# speedup vs baseline: 1.3628x; 1.3628x over previous
import functools

import jax
import jax.numpy as jnp
from jax.experimental import pallas as pl
from jax.experimental.pallas import tpu as pltpu

N = 2048
NUM_EDGE = 5
NUM_CHANNELS = 2
IN_DIM = 256
HIDDEN = 64
NUM_CLASS = 8
WIDE = 384
BM = 256


def _first_kernel(filt_ref, a_ref, t0_ref, t_ref, q1_ref, q0_ref):
    ab = [a_ref[r].astype(jnp.bfloat16) for r in range(NUM_EDGE)]
    B = [jnp.dot(ab[r], t0_ref[...], preferred_element_type=jnp.float32)
         for r in range(NUM_EDGE)]
    for c in range(NUM_CHANNELS):
        t = filt_ref[2, c, 0] * B[0]
        for r in range(1, NUM_EDGE):
            t = t + filt_ref[2, c, r] * B[r]
        t_ref[c] = t.astype(jnp.bfloat16)
        for s, qref in ((1, q1_ref), (0, q0_ref)):
            acc = filt_ref[s, c, 0].astype(jnp.bfloat16) * ab[0]
            for r in range(1, NUM_EDGE):
                acc = acc + filt_ref[s, c, r].astype(jnp.bfloat16) * ab[r]
            qref[c] = acc


def _stage_kernel(q_ref, t_ref, out_ref):
    for c in range(NUM_CHANNELS):
        out_ref[c] = jnp.dot(q_ref[c], t_ref[c],
                             preferred_element_type=jnp.float32).astype(jnp.bfloat16)


def _final_kernel(q_ref, t_ref, gw_ref, gb_ref, l1w_ref, l1b_ref,
                  l2w_ref, l2b_ref, y_ref):
    xs = []
    for c in range(NUM_CHANNELS):
        v = jnp.dot(q_ref[c], t_ref[c],
                    preferred_element_type=jnp.float32)
        num = v[:, :IN_DIM]
        e = v[:, IN_DIM:IN_DIM + 1]
        agg = num / jnp.where(e == 0.0, 1.0, e)
        x = jnp.dot(agg, gw_ref[...], preferred_element_type=jnp.float32)
        xs.append(jnp.maximum(x + gb_ref[...], 0.0))
    z = (jnp.dot(xs[0], l1w_ref[:HIDDEN], preferred_element_type=jnp.float32)
         + jnp.dot(xs[1], l1w_ref[HIDDEN:], preferred_element_type=jnp.float32)
         + l1b_ref[...])
    z = jnp.maximum(z, 0.0)
    y_ref[...] = (jnp.dot(z, l2w_ref[...], preferred_element_type=jnp.float32)
                  + l2b_ref[...])


def _first(filt, A, t0):
    return pl.pallas_call(
        _first_kernel,
        grid=(N // BM,),
        in_specs=[
            pl.BlockSpec(memory_space=pltpu.SMEM),
            pl.BlockSpec((NUM_EDGE, BM, N), lambda i: (0, i, 0)),
            pl.BlockSpec((N, WIDE), lambda i: (0, 0)),
        ],
        out_specs=[
            pl.BlockSpec((NUM_CHANNELS, BM, WIDE), lambda i: (0, i, 0)),
            pl.BlockSpec((NUM_CHANNELS, BM, N), lambda i: (0, i, 0)),
            pl.BlockSpec((NUM_CHANNELS, BM, N), lambda i: (0, i, 0)),
        ],
        out_shape=[
            jax.ShapeDtypeStruct((NUM_CHANNELS, N, WIDE), jnp.bfloat16),
            jax.ShapeDtypeStruct((NUM_CHANNELS, N, N), jnp.bfloat16),
            jax.ShapeDtypeStruct((NUM_CHANNELS, N, N), jnp.bfloat16),
        ],
    )(filt, A, t0)


def _stage(Q, t):
    return pl.pallas_call(
        _stage_kernel,
        grid=(N // BM,),
        in_specs=[
            pl.BlockSpec((NUM_CHANNELS, BM, N), lambda i: (0, i, 0)),
            pl.BlockSpec((NUM_CHANNELS, N, WIDE), lambda i: (0, 0, 0)),
        ],
        out_specs=pl.BlockSpec((NUM_CHANNELS, BM, WIDE), lambda i: (0, i, 0)),
        out_shape=jax.ShapeDtypeStruct((NUM_CHANNELS, N, WIDE), jnp.bfloat16),
    )(Q, t)


def _final(Q, t, gcn_w, gcn_b, lin1_w, lin1_b, lin2_w, lin2_b):
    small = lambda shp: pl.BlockSpec(shp, lambda i: tuple(0 for _ in shp))
    return pl.pallas_call(
        _final_kernel,
        grid=(N // BM,),
        in_specs=[
            pl.BlockSpec((NUM_CHANNELS, BM, N), lambda i: (0, i, 0)),
            pl.BlockSpec((NUM_CHANNELS, N, WIDE), lambda i: (0, 0, 0)),
            small((IN_DIM, HIDDEN)),
            small((1, HIDDEN)),
            small((NUM_CHANNELS * HIDDEN, HIDDEN)),
            small((1, HIDDEN)),
            small((HIDDEN, NUM_CLASS)),
            small((1, NUM_CLASS)),
        ],
        out_specs=pl.BlockSpec((BM, NUM_CLASS), lambda i: (i, 0)),
        out_shape=jax.ShapeDtypeStruct((N, NUM_CLASS), jnp.float32),
    )(Q, t, gcn_w, gcn_b, lin1_w, lin1_b, lin2_w, lin2_b)


def kernel(A, h, W_conv, gcn_w, gcn_b, lin1_w, lin1_b, lin2_w, lin2_b):
    filt = jax.nn.softmax(W_conv, axis=2)
    t0 = jnp.concatenate(
        [h, jnp.ones((N, 1), jnp.float32), jnp.zeros((N, WIDE - IN_DIM - 1), jnp.float32)],
        axis=1).astype(jnp.bfloat16)
    t, Q1, Q0 = _first(filt, A, t0)
    t = _stage(Q1, t)
    return _final(Q0, t,
                  gcn_w, gcn_b.reshape(1, HIDDEN),
                  lin1_w, lin1_b.reshape(1, HIDDEN),
                  lin2_w, lin2_b.reshape(1, NUM_CLASS))


# speedup vs baseline: 1.3658x; 1.0022x over previous
import functools

import jax
import jax.numpy as jnp
from jax.experimental import pallas as pl
from jax.experimental.pallas import tpu as pltpu

N = 2048
NUM_EDGE = 5
NUM_CHANNELS = 2
IN_DIM = 256
HIDDEN = 64
NUM_CLASS = 8
WIDE = 384
BM = 256


def _first_kernel(filt_ref, a_ref, t0_ref, t_ref, q1_ref, q0_ref):
    ab = [a_ref[r].astype(jnp.bfloat16) for r in range(NUM_EDGE)]
    B = [jnp.dot(ab[r], t0_ref[...], preferred_element_type=jnp.float32)
         for r in range(NUM_EDGE)]
    for c in range(NUM_CHANNELS):
        t = filt_ref[2, c, 0] * B[0]
        for r in range(1, NUM_EDGE):
            t = t + filt_ref[2, c, r] * B[r]
        t_ref[c] = t.astype(jnp.bfloat16)
        for s, qref in ((1, q1_ref), (0, q0_ref)):
            acc = filt_ref[s, c, 0].astype(jnp.bfloat16) * ab[0]
            for r in range(1, NUM_EDGE):
                acc = acc + filt_ref[s, c, r].astype(jnp.bfloat16) * ab[r]
            qref[c] = acc


def _stage_kernel(q_ref, t_ref, out_ref):
    for c in range(NUM_CHANNELS):
        out_ref[c] = jnp.dot(q_ref[c], t_ref[c],
                             preferred_element_type=jnp.float32).astype(jnp.bfloat16)


def _final_kernel(q_ref, t_ref, gw_ref, gb_ref, l1w_ref, l1b_ref,
                  l2w_ref, l2b_ref, y_ref):
    xs = []
    for c in range(NUM_CHANNELS):
        v = jnp.dot(q_ref[c], t_ref[c],
                    preferred_element_type=jnp.float32)
        num = v[:, :IN_DIM]
        e = v[:, IN_DIM:IN_DIM + 1]
        agg = num * (1.0 / jnp.where(e == 0.0, 1.0, e))
        x = jnp.dot(agg, gw_ref[...], preferred_element_type=jnp.float32)
        xs.append(jnp.maximum(x + gb_ref[...], 0.0))
    z = (jnp.dot(xs[0], l1w_ref[:HIDDEN], preferred_element_type=jnp.float32)
         + jnp.dot(xs[1], l1w_ref[HIDDEN:], preferred_element_type=jnp.float32)
         + l1b_ref[...])
    z = jnp.maximum(z, 0.0)
    y_ref[...] = (jnp.dot(z, l2w_ref[...], preferred_element_type=jnp.float32)
                  + l2b_ref[...])


_PAR = pltpu.CompilerParams(dimension_semantics=("parallel",))


def _first(filt, A, t0):
    return pl.pallas_call(
        _first_kernel,
        grid=(N // BM,),
        compiler_params=_PAR,
        in_specs=[
            pl.BlockSpec(memory_space=pltpu.SMEM),
            pl.BlockSpec((NUM_EDGE, BM, N), lambda i: (0, i, 0)),
            pl.BlockSpec((N, WIDE), lambda i: (0, 0)),
        ],
        out_specs=[
            pl.BlockSpec((NUM_CHANNELS, BM, WIDE), lambda i: (0, i, 0)),
            pl.BlockSpec((NUM_CHANNELS, BM, N), lambda i: (0, i, 0)),
            pl.BlockSpec((NUM_CHANNELS, BM, N), lambda i: (0, i, 0)),
        ],
        out_shape=[
            jax.ShapeDtypeStruct((NUM_CHANNELS, N, WIDE), jnp.bfloat16),
            jax.ShapeDtypeStruct((NUM_CHANNELS, N, N), jnp.bfloat16),
            jax.ShapeDtypeStruct((NUM_CHANNELS, N, N), jnp.bfloat16),
        ],
    )(filt, A, t0)


def _stage(Q, t):
    return pl.pallas_call(
        _stage_kernel,
        grid=(N // BM,),
        compiler_params=_PAR,
        in_specs=[
            pl.BlockSpec((NUM_CHANNELS, BM, N), lambda i: (0, i, 0)),
            pl.BlockSpec((NUM_CHANNELS, N, WIDE), lambda i: (0, 0, 0)),
        ],
        out_specs=pl.BlockSpec((NUM_CHANNELS, BM, WIDE), lambda i: (0, i, 0)),
        out_shape=jax.ShapeDtypeStruct((NUM_CHANNELS, N, WIDE), jnp.bfloat16),
    )(Q, t)


def _final(Q, t, gcn_w, gcn_b, lin1_w, lin1_b, lin2_w, lin2_b):
    small = lambda shp: pl.BlockSpec(shp, lambda i: tuple(0 for _ in shp))
    return pl.pallas_call(
        _final_kernel,
        grid=(N // BM,),
        compiler_params=_PAR,
        in_specs=[
            pl.BlockSpec((NUM_CHANNELS, BM, N), lambda i: (0, i, 0)),
            pl.BlockSpec((NUM_CHANNELS, N, WIDE), lambda i: (0, 0, 0)),
            small((IN_DIM, HIDDEN)),
            small((1, HIDDEN)),
            small((NUM_CHANNELS * HIDDEN, HIDDEN)),
            small((1, HIDDEN)),
            small((HIDDEN, NUM_CLASS)),
            small((1, NUM_CLASS)),
        ],
        out_specs=pl.BlockSpec((BM, NUM_CLASS), lambda i: (i, 0)),
        out_shape=jax.ShapeDtypeStruct((N, NUM_CLASS), jnp.float32),
    )(Q, t, gcn_w, gcn_b, lin1_w, lin1_b, lin2_w, lin2_b)


def kernel(A, h, W_conv, gcn_w, gcn_b, lin1_w, lin1_b, lin2_w, lin2_b):
    filt = jax.nn.softmax(W_conv, axis=2)
    t0 = jnp.concatenate(
        [h, jnp.ones((N, 1), jnp.float32), jnp.zeros((N, WIDE - IN_DIM - 1), jnp.float32)],
        axis=1).astype(jnp.bfloat16)
    t, Q1, Q0 = _first(filt, A, t0)
    t = _stage(Q1, t)
    return _final(Q0, t,
                  gcn_w, gcn_b.reshape(1, HIDDEN),
                  lin1_w, lin1_b.reshape(1, HIDDEN),
                  lin2_w, lin2_b.reshape(1, NUM_CLASS))
